# Initial kernel scaffold; baseline (speedup 1.0000x reference)
import jax
import jax.numpy as jnp
from jax.experimental import pallas as pl


def kernel(x, Wr, br, sgW, svW, soW, sob, egW, evW, eoW, eob):
    raise NotImplementedError("write your pallas kernel here")


# speedup vs baseline: 1.5143x; 1.5143x over previous
import functools

import jax
import jax.numpy as jnp
from jax.experimental import pallas as pl
from jax.experimental.pallas import tpu as pltpu

B = 1
S = 2048
D_MODEL = 1024
HIDDEN = 1024
OUT_DIM = 1024
NUM_EXPERTS = 8
TOP_K = 2

TS = 1024
NT = S // TS


def _moe_kernel(x_ref, wr_ref, br_ref, gw_ref, vw_ref, ow_ref, ob_ref,
                out_ref, logits_ref, topk_ref, sw_ref):
    j = pl.program_id(1)
    xt = x_ref[...]

    @pl.when(j == 0)
    def _router():
        logits = jnp.dot(xt, wr_ref[...], preferred_element_type=jnp.float32)
        logits = logits + br_ref[...]
        logits_ref[...] = logits
        m = jnp.max(logits, axis=1, keepdims=True)
        e = jnp.exp(logits - m)
        gw = e / jnp.sum(e, axis=1, keepdims=True)
        lane = jax.lax.broadcasted_iota(jnp.int32, (TS, NUM_EXPERTS), 1)
        i1 = jnp.min(jnp.where(logits == m, lane, NUM_EXPERTS), axis=1,
                     keepdims=True)
        masked = jnp.where(lane == i1, -jnp.inf, logits)
        m2 = jnp.max(masked, axis=1, keepdims=True)
        i2 = jnp.min(jnp.where(masked == m2, lane, NUM_EXPERTS), axis=1,
                     keepdims=True)
        w1 = jnp.sum(jnp.where(lane == i1, gw, 0.0), axis=1, keepdims=True)
        w2 = jnp.sum(jnp.where(lane == i2, gw, 0.0), axis=1, keepdims=True)
        s = w1 + w2
        sw_ref[...] = (jnp.where(lane == i1, w1 / s, 0.0)
                       + jnp.where(lane == i2, w2 / s, 0.0))
        topk_ref[...] = jnp.concatenate([i1, i2], axis=1)
        out_ref[...] = jnp.zeros_like(out_ref)

    g = jnp.dot(xt, gw_ref[0], preferred_element_type=jnp.float32)
    v = jnp.dot(xt, vw_ref[0], preferred_element_type=jnp.float32)
    h = (g * jax.lax.logistic(g)) * v
    y = jnp.dot(h, ow_ref[0], preferred_element_type=jnp.float32) + ob_ref[0]

    lane = jax.lax.broadcasted_iota(jnp.int32, (TS, NUM_EXPERTS), 1)
    w_col = jnp.sum(jnp.where(lane == j, sw_ref[...], 0.0), axis=1,
                    keepdims=True)
    w_col = jnp.where(j == NUM_EXPERTS, 1.0, w_col)
    out_ref[...] += w_col * y


@jax.jit
def kernel(x, Wr, br, sgW, svW, soW, sob, egW, evW, eoW, eob):
    x2 = x.reshape(S, D_MODEL)
    gW_all = jnp.concatenate([egW, sgW[None]], axis=0)
    vW_all = jnp.concatenate([evW, svW[None]], axis=0)
    oW_all = jnp.concatenate([eoW, soW[None]], axis=0)
    ob_all = jnp.concatenate([eob, sob[None]], axis=0).reshape(
        NUM_EXPERTS + 1, 1, OUT_DIM)

    E1 = NUM_EXPERTS + 1
    out, logits, topk = pl.pallas_call(
        _moe_kernel,
        grid=(NT, E1),
        in_specs=[
            pl.BlockSpec((TS, D_MODEL), lambda i, j: (i, 0)),
            pl.BlockSpec((D_MODEL, NUM_EXPERTS), lambda i, j: (0, 0)),
            pl.BlockSpec((1, NUM_EXPERTS), lambda i, j: (0, 0)),
            pl.BlockSpec((1, D_MODEL, HIDDEN), lambda i, j: (j, 0, 0)),
            pl.BlockSpec((1, D_MODEL, HIDDEN), lambda i, j: (j, 0, 0)),
            pl.BlockSpec((1, HIDDEN, OUT_DIM), lambda i, j: (j, 0, 0)),
            pl.BlockSpec((1, 1, OUT_DIM), lambda i, j: (j, 0, 0)),
        ],
        out_specs=[
            pl.BlockSpec((TS, OUT_DIM), lambda i, j: (i, 0)),
            pl.BlockSpec((TS, NUM_EXPERTS), lambda i, j: (i, 0)),
            pl.BlockSpec((TS, TOP_K), lambda i, j: (i, 0)),
        ],
        out_shape=[
            jax.ShapeDtypeStruct((S, OUT_DIM), jnp.float32),
            jax.ShapeDtypeStruct((S, NUM_EXPERTS), jnp.float32),
            jax.ShapeDtypeStruct((S, TOP_K), jnp.int32),
        ],
        scratch_shapes=[
            pltpu.VMEM((TS, NUM_EXPERTS), jnp.float32),
        ],
    )(x2, Wr, br.reshape(1, NUM_EXPERTS), gW_all, vW_all, oW_all, ob_all)

    return (out.reshape(B, S, OUT_DIM),
            logits.reshape(B, S, NUM_EXPERTS),
            topk.reshape(B, S, TOP_K))
